# baseline (speedup 1.0000x reference)
import functools

import jax
import jax.numpy as jnp
from jax import lax
from jax.experimental import pallas as pl
from jax.experimental.pallas import tpu as pltpu

_IOU_THRESH = 0.5
_NEG_POS_RATIO = 3
_V0 = 0.1
_V1 = 0.2
_C = 21
_N = 32
_P = 8732
_L = 128
_R = 69
_PP = _R * _L
_G = 1


def _one_image(X, Lc, T, px, py, pw, ph, pf_x1, pf_y1, pf_x2, pf_y2,
               p_area, flat, valid):
    f32 = jnp.float32
    i32 = jnp.int32

    bto = jnp.full((_R, _L), -1.0, dtype=f32)
    bti = jnp.zeros((_R, _L), dtype=i32)
    forced_any = jnp.zeros((_R, _L), dtype=jnp.bool_)
    forced_n = jnp.zeros((_R, _L), dtype=i32)
    for n in range(_N):
        tx1 = T[n, 0]
        ty1 = T[n, 1]
        tx2 = T[n, 2]
        ty2 = T[n, 3]
        iw = jnp.maximum(jnp.minimum(pf_x2, tx2) - jnp.maximum(pf_x1, tx1), 0.0)
        ih = jnp.maximum(jnp.minimum(pf_y2, ty2) - jnp.maximum(pf_y1, ty1), 0.0)
        inter = iw * ih
        t_area = (tx2 - tx1) * (ty2 - ty1)
        iou = inter / (t_area + p_area - inter)
        iou = jnp.where(valid, iou, -1.0)
        upd = iou > bto
        bti = jnp.where(upd, n, bti)
        bto = jnp.where(upd, iou, bto)
        mx = jnp.max(iou, axis=(0, 1), keepdims=True)
        bpi_n = jnp.min(jnp.where(iou == mx, flat, jnp.int32(2 ** 30)),
                        axis=(0, 1), keepdims=True)
        eq = flat == bpi_n
        forced_any = forced_any | eq
        forced_n = jnp.where(eq, n, forced_n)

    bti = jnp.where(forced_any, forced_n, bti)
    bto = jnp.where(forced_any, 2.0, bto)

    mx1 = jnp.zeros((_R, _L), dtype=f32)
    my1 = jnp.zeros((_R, _L), dtype=f32)
    mx2 = jnp.zeros((_R, _L), dtype=f32)
    my2 = jnp.zeros((_R, _L), dtype=f32)
    lab = jnp.zeros((_R, _L), dtype=f32)
    for n in range(_N):
        m = bti == n
        mx1 = jnp.where(m, T[n, 0], mx1)
        my1 = jnp.where(m, T[n, 1], my1)
        mx2 = jnp.where(m, T[n, 2], mx2)
        my2 = jnp.where(m, T[n, 3], my2)
        lab = jnp.where(m, T[n, 4], lab)

    tcls = jnp.where(bto < _IOU_THRESH, 0.0, lab).astype(i32)
    pos = tcls > 0
    npos = jnp.sum(pos.astype(i32), axis=(0, 1), keepdims=True)
    K = npos * _NEG_POS_RATIO

    m = X[0]
    for c in range(1, _C):
        m = jnp.maximum(m, X[c])
    s = jnp.zeros((_R, _L), dtype=f32)
    for c in range(_C):
        s = s + jnp.exp(X[c] - m)
    lse = jnp.log(s) + m
    xt = jnp.zeros((_R, _L), dtype=f32)
    for c in range(_C):
        xt = jnp.where(tcls == c, X[c], xt)
    ce = lse - xt
    ce_pos_sum = jnp.sum(jnp.where(pos, ce, 0.0), axis=(0, 1), keepdims=True)
    ce_m = jnp.where(pos | jnp.logical_not(valid), 0.0, ce)

    ibits = lax.bitcast_convert_type(ce_m, i32)
    cur = jnp.zeros((1, 1), dtype=i32)
    for i in range(31):
        cand = cur | jnp.int32(1 << (30 - i))
        cnt = jnp.sum((ibits >= cand).astype(i32), axis=(0, 1), keepdims=True)
        cur = jnp.where(cnt >= K, cand, cur)
    cnt_gt = jnp.sum((ibits > cur).astype(i32), axis=(0, 1), keepdims=True)
    sum_gt = jnp.sum(jnp.where(ibits > cur, ce_m, 0.0),
                     axis=(0, 1), keepdims=True)
    tval = lax.bitcast_convert_type(cur, f32)
    topk = jnp.where(
        K > 0, sum_gt + (K - cnt_gt).astype(f32) * tval, 0.0)

    pw_s = jnp.where(valid, pw, 1.0)
    ph_s = jnp.where(valid, ph, 1.0)
    g_cx = ((mx1 + mx2) * 0.5 - px) / (_V0 * pw_s)
    g_cy = ((my1 + my2) * 0.5 - py) / (_V0 * ph_s)
    g_w = jnp.log(jnp.maximum((mx2 - mx1) / pw_s, 1e-8)) * (1.0 / _V1)
    g_h = jnp.log(jnp.maximum((my2 - my1) / ph_s, 1e-8)) * (1.0 / _V1)
    loc_sum = jnp.zeros((1, 1), dtype=f32)
    for d, g in ((0, g_cx), (1, g_cy), (2, g_w), (3, g_h)):
        diff = Lc[d] - g
        ad = jnp.abs(diff)
        sl1 = jnp.where(ad < 1.0, 0.5 * diff * diff, ad - 0.5)
        loc_sum = loc_sum + jnp.sum(jnp.where(pos, sl1, 0.0),
                                    axis=(0, 1), keepdims=True)

    return jnp.concatenate(
        [
            jnp.broadcast_to(loc_sum, (1, _L)),
            jnp.broadcast_to(ce_pos_sum, (1, _L)),
            jnp.broadcast_to(topk, (1, _L)),
            jnp.broadcast_to(npos.astype(f32), (1, _L)),
            jnp.zeros((4, _L), dtype=f32),
        ],
        axis=0,
    )


def _loss_kernel(conf_ref, loc_ref, priors_ref, targets_ref, out_ref):
    i32 = jnp.int32

    px = priors_ref[0]
    py = priors_ref[1]
    pw = priors_ref[2]
    ph = priors_ref[3]
    pf_x1 = px - pw * 0.5
    pf_y1 = py - ph * 0.5
    pf_x2 = px + pw * 0.5
    pf_y2 = py + ph * 0.5
    p_area = pw * ph

    ir = lax.broadcasted_iota(i32, (_R, _L), 0)
    il = lax.broadcasted_iota(i32, (_R, _L), 1)
    flat = ir * _L + il
    valid = flat < _P

    for g in range(_G):
        out_ref[g] = _one_image(
            conf_ref[g], loc_ref[g], targets_ref[g],
            px, py, pw, ph, pf_x1, pf_y1, pf_x2, pf_y2,
            p_area, flat, valid)


@functools.partial(jax.jit, static_argnames=())
def kernel(loc_data, conf_data, priors, targets):
    B = conf_data.shape[0]
    pad = _PP - _P
    conf_p = jnp.pad(
        jnp.transpose(conf_data, (0, 2, 1)), ((0, 0), (0, 0), (0, pad))
    ).reshape(B, _C, _R, _L)
    loc_p = jnp.pad(
        jnp.transpose(loc_data, (0, 2, 1)), ((0, 0), (0, 0), (0, pad))
    ).reshape(B, 4, _R, _L)
    priors_p = jnp.pad(priors.T, ((0, 0), (0, pad))).reshape(4, _R, _L)

    out = pl.pallas_call(
        _loss_kernel,
        grid=(B // _G,),
        in_specs=[
            pl.BlockSpec((_G, _C, _R, _L), lambda b: (b, 0, 0, 0)),
            pl.BlockSpec((_G, 4, _R, _L), lambda b: (b, 0, 0, 0)),
            pl.BlockSpec((4, _R, _L), lambda b: (0, 0, 0)),
            pl.BlockSpec((_G, _N, 5), lambda b: (b, 0, 0)),
        ],
        out_specs=pl.BlockSpec((_G, 8, _L), lambda b: (b, 0, 0)),
        out_shape=jax.ShapeDtypeStruct((B, 8, _L), jnp.float32),
        compiler_params=pltpu.CompilerParams(
            dimension_semantics=("arbitrary",),
        ),
    )(conf_p, loc_p, priors_p, targets)

    sums = out[:, :4, 0]
    num_pos = jnp.sum(sums[:, 3])
    loc_loss = jnp.sum(sums[:, 0]) / num_pos
    conf_loss = (jnp.sum(sums[:, 1]) + jnp.sum(sums[:, 2])) / num_pos
    return (loc_loss, conf_loss)


# speedup vs baseline: 2.1883x; 2.1883x over previous
import functools

import jax
import jax.numpy as jnp
from jax import lax
from jax.experimental import pallas as pl
from jax.experimental.pallas import tpu as pltpu

_IOU_THRESH = 0.5
_NEG_POS_RATIO = 3
_V0 = 0.1
_V1 = 0.2
_C = 21
_N = 32
_P = 8732
_L = 128
_R = 69
_PP = _R * _L
_G = 1


def _one_image(X, Lc, T, px, py, pw, ph, pf_x1, pf_y1, pf_x2, pf_y2,
               p_area, flat, valid):
    f32 = jnp.float32
    i32 = jnp.int32

    bto = jnp.full((_R, _L), -1.0, dtype=f32)
    bti = jnp.zeros((_R, _L), dtype=i32)
    ious = []
    colmaxes = []
    for n in range(_N):
        tx1 = T[n, 0]
        ty1 = T[n, 1]
        tx2 = T[n, 2]
        ty2 = T[n, 3]
        iw = jnp.maximum(jnp.minimum(pf_x2, tx2) - jnp.maximum(pf_x1, tx1), 0.0)
        ih = jnp.maximum(jnp.minimum(pf_y2, ty2) - jnp.maximum(pf_y1, ty1), 0.0)
        inter = iw * ih
        t_area = (tx2 - tx1) * (ty2 - ty1)
        iou = inter / (t_area + p_area - inter)
        iou = jnp.where(valid, iou, -1.0)
        upd = iou > bto
        bti = jnp.where(upd, n, bti)
        bto = jnp.where(upd, iou, bto)
        ious.append(iou)
        colmaxes.append(jnp.max(iou, axis=0, keepdims=True))

    mxv = jnp.max(jnp.concatenate(colmaxes, axis=0), axis=1,
                  keepdims=True)
    colidx = []
    for n in range(_N):
        cm = jnp.where(ious[n] == mxv[n, 0], flat, jnp.int32(2 ** 30))
        colidx.append(jnp.min(cm, axis=0, keepdims=True))
    bpiv = jnp.min(jnp.concatenate(colidx, axis=0), axis=1,
                   keepdims=True)

    forced_any = jnp.zeros((_R, _L), dtype=jnp.bool_)
    forced_n = jnp.zeros((_R, _L), dtype=i32)
    for n in range(_N):
        eq = flat == bpiv[n, 0]
        forced_any = forced_any | eq
        forced_n = jnp.where(eq, n, forced_n)

    bti = jnp.where(forced_any, forced_n, bti)
    bto = jnp.where(forced_any, 2.0, bto)

    mx1 = jnp.zeros((_R, _L), dtype=f32)
    my1 = jnp.zeros((_R, _L), dtype=f32)
    mx2 = jnp.zeros((_R, _L), dtype=f32)
    my2 = jnp.zeros((_R, _L), dtype=f32)
    lab = jnp.zeros((_R, _L), dtype=f32)
    for n in range(_N):
        m = bti == n
        mx1 = jnp.where(m, T[n, 0], mx1)
        my1 = jnp.where(m, T[n, 1], my1)
        mx2 = jnp.where(m, T[n, 2], mx2)
        my2 = jnp.where(m, T[n, 3], my2)
        lab = jnp.where(m, T[n, 4], lab)

    tcls = jnp.where(bto < _IOU_THRESH, 0.0, lab).astype(i32)
    pos = tcls > 0
    npos = jnp.sum(pos.astype(i32), axis=(0, 1), keepdims=True)
    K = npos * _NEG_POS_RATIO

    m = X[0]
    for c in range(1, _C):
        m = jnp.maximum(m, X[c])
    s = jnp.zeros((_R, _L), dtype=f32)
    for c in range(_C):
        s = s + jnp.exp(X[c] - m)
    lse = jnp.log(s) + m
    xt = jnp.zeros((_R, _L), dtype=f32)
    for c in range(_C):
        xt = jnp.where(tcls == c, X[c], xt)
    ce = lse - xt
    ce_pos_sum = jnp.sum(jnp.where(pos, ce, 0.0), axis=(0, 1), keepdims=True)
    ce_m = jnp.where(pos | jnp.logical_not(valid), 0.0, ce)

    ibits = lax.bitcast_convert_type(ce_m, i32)
    cur = jnp.zeros((1, 1), dtype=i32)
    for i in range(31):
        cand = cur | jnp.int32(1 << (30 - i))
        cnt = jnp.sum((ibits >= cand).astype(i32), axis=(0, 1), keepdims=True)
        cur = jnp.where(cnt >= K, cand, cur)
    cnt_gt = jnp.sum((ibits > cur).astype(i32), axis=(0, 1), keepdims=True)
    sum_gt = jnp.sum(jnp.where(ibits > cur, ce_m, 0.0),
                     axis=(0, 1), keepdims=True)
    tval = lax.bitcast_convert_type(cur, f32)
    topk = jnp.where(
        K > 0, sum_gt + (K - cnt_gt).astype(f32) * tval, 0.0)

    pw_s = jnp.where(valid, pw, 1.0)
    ph_s = jnp.where(valid, ph, 1.0)
    g_cx = ((mx1 + mx2) * 0.5 - px) / (_V0 * pw_s)
    g_cy = ((my1 + my2) * 0.5 - py) / (_V0 * ph_s)
    g_w = jnp.log(jnp.maximum((mx2 - mx1) / pw_s, 1e-8)) * (1.0 / _V1)
    g_h = jnp.log(jnp.maximum((my2 - my1) / ph_s, 1e-8)) * (1.0 / _V1)
    loc_sum = jnp.zeros((1, 1), dtype=f32)
    for d, g in ((0, g_cx), (1, g_cy), (2, g_w), (3, g_h)):
        diff = Lc[d] - g
        ad = jnp.abs(diff)
        sl1 = jnp.where(ad < 1.0, 0.5 * diff * diff, ad - 0.5)
        loc_sum = loc_sum + jnp.sum(jnp.where(pos, sl1, 0.0),
                                    axis=(0, 1), keepdims=True)

    return jnp.concatenate(
        [
            jnp.broadcast_to(loc_sum, (1, _L)),
            jnp.broadcast_to(ce_pos_sum, (1, _L)),
            jnp.broadcast_to(topk, (1, _L)),
            jnp.broadcast_to(npos.astype(f32), (1, _L)),
            jnp.zeros((4, _L), dtype=f32),
        ],
        axis=0,
    )


def _loss_kernel(conf_ref, loc_ref, priors_ref, targets_ref, out_ref):
    i32 = jnp.int32

    px = priors_ref[0]
    py = priors_ref[1]
    pw = priors_ref[2]
    ph = priors_ref[3]
    pf_x1 = px - pw * 0.5
    pf_y1 = py - ph * 0.5
    pf_x2 = px + pw * 0.5
    pf_y2 = py + ph * 0.5
    p_area = pw * ph

    ir = lax.broadcasted_iota(i32, (_R, _L), 0)
    il = lax.broadcasted_iota(i32, (_R, _L), 1)
    flat = ir * _L + il
    valid = flat < _P

    for g in range(_G):
        out_ref[g] = _one_image(
            conf_ref[g], loc_ref[g], targets_ref[g],
            px, py, pw, ph, pf_x1, pf_y1, pf_x2, pf_y2,
            p_area, flat, valid)


@functools.partial(jax.jit, static_argnames=())
def kernel(loc_data, conf_data, priors, targets):
    B = conf_data.shape[0]
    pad = _PP - _P
    conf_p = jnp.pad(
        jnp.transpose(conf_data, (0, 2, 1)), ((0, 0), (0, 0), (0, pad))
    ).reshape(B, _C, _R, _L)
    loc_p = jnp.pad(
        jnp.transpose(loc_data, (0, 2, 1)), ((0, 0), (0, 0), (0, pad))
    ).reshape(B, 4, _R, _L)
    priors_p = jnp.pad(priors.T, ((0, 0), (0, pad))).reshape(4, _R, _L)

    out = pl.pallas_call(
        _loss_kernel,
        grid=(B // _G,),
        in_specs=[
            pl.BlockSpec((_G, _C, _R, _L), lambda b: (b, 0, 0, 0)),
            pl.BlockSpec((_G, 4, _R, _L), lambda b: (b, 0, 0, 0)),
            pl.BlockSpec((4, _R, _L), lambda b: (0, 0, 0)),
            pl.BlockSpec((_G, _N, 5), lambda b: (b, 0, 0)),
        ],
        out_specs=pl.BlockSpec((_G, 8, _L), lambda b: (b, 0, 0)),
        out_shape=jax.ShapeDtypeStruct((B, 8, _L), jnp.float32),
        compiler_params=pltpu.CompilerParams(
            dimension_semantics=("arbitrary",),
        ),
    )(conf_p, loc_p, priors_p, targets)

    sums = out[:, :4, 0]
    num_pos = jnp.sum(sums[:, 3])
    loc_loss = jnp.sum(sums[:, 0]) / num_pos
    conf_loss = (jnp.sum(sums[:, 1]) + jnp.sum(sums[:, 2])) / num_pos
    return (loc_loss, conf_loss)
